# baseline (speedup 1.0000x reference)
import functools

import jax
import jax.numpy as jnp
from jax import lax
from jax.experimental import pallas as pl
from jax.experimental.pallas import tpu as pltpu
from jax.experimental.pallas import tpu_sc as plsc

EMB = 64
HID = 256
NC = 2
NS = 16
NW = NC * NS
CHUNK = 128


def _sc_gather_body(uidx_hbm, iidx_hbm, uemb, iemb, uout, iout,
                    uidx, iidx, ubuf, ibuf, sem, *, nchunk, bpw):
    wid = lax.axis_index("s") * NC + lax.axis_index("c")
    base = wid * bpw
    pltpu.sync_copy(uidx_hbm.at[wid], uidx)
    pltpu.sync_copy(iidx_hbm.at[wid], iidx)
    npass = nchunk // 2
    for p in range(npass):
        cps = []
        for j in range(2):
            c = 2 * p + j
            cps.append(pltpu.async_copy(
                uemb.at[uidx.at[c]], ubuf.at[pl.ds(j * CHUNK, CHUNK)], sem))
            cps.append(pltpu.async_copy(
                iemb.at[iidx.at[c]], ibuf.at[pl.ds(j * CHUNK, CHUNK)], sem))
        for cp in cps:
            cp.wait()
        off = base + p * 2 * CHUNK
        pltpu.sync_copy(ubuf, uout.at[pl.ds(off, 2 * CHUNK)])
        pltpu.sync_copy(ibuf, iout.at[pl.ds(off, 2 * CHUNK)])


def _mlp_body(gu, gi, uh, ih, w1uu, w1ii, b1, w2t, b2, o):
    bm = gu.shape[0]
    lane_half = jax.lax.broadcasted_iota(jnp.int32, (bm, 2 * EMB), 1) // EMB
    mu = jnp.where(lane_half == uh[...], gu[...], 0.0)
    mi = jnp.where(lane_half == ih[...], gi[...], 0.0)
    x = jnp.dot(mu, w1uu[...], preferred_element_type=jnp.float32)
    x = x + jnp.dot(mi, w1ii[...], preferred_element_type=jnp.float32)
    x = jnp.maximum(x + b1[...], 0.0)
    o[...] = jnp.sum(x * w2t[...], axis=1, keepdims=True) + b2[...]


def kernel(user_ids, item_ids, user_emb, item_emb, W1, b1, W2, b2):
    B = user_ids.shape[0]
    bpw = B // NW
    nchunk = bpw // CHUNK
    uids = user_ids.astype(jnp.int32)
    iids = item_ids.astype(jnp.int32)
    uidx_r = (uids >> 1).reshape(NW, nchunk, CHUNK)
    iidx_r = (iids >> 1).reshape(NW, nchunk, CHUNK)
    uemb2 = user_emb.reshape(user_emb.shape[0] // 2, 2 * EMB)
    iemb2 = item_emb.reshape(item_emb.shape[0] // 2, 2 * EMB)

    gather = pl.kernel(
        functools.partial(_sc_gather_body, nchunk=nchunk, bpw=bpw),
        out_type=(jax.ShapeDtypeStruct((B, 2 * EMB), jnp.float32),
                  jax.ShapeDtypeStruct((B, 2 * EMB), jnp.float32)),
        mesh=plsc.VectorSubcoreMesh(core_axis_name="c", subcore_axis_name="s"),
        scratch_types=[
            pltpu.VMEM((nchunk, CHUNK), jnp.int32),
            pltpu.VMEM((nchunk, CHUNK), jnp.int32),
            pltpu.VMEM((2 * CHUNK, 2 * EMB), jnp.float32),
            pltpu.VMEM((2 * CHUNK, 2 * EMB), jnp.float32),
            pltpu.SemaphoreType.DMA,
        ],
    )
    gu, gi = gather(uidx_r, iidx_r, uemb2, iemb2)

    W1uu = jnp.concatenate([W1[:EMB], W1[:EMB]], axis=0)
    W1ii = jnp.concatenate([W1[EMB:], W1[EMB:]], axis=0)
    uh = (uids & 1).reshape(B, 1)
    ih = (iids & 1).reshape(B, 1)

    BM = 2048
    out = pl.pallas_call(
        _mlp_body,
        grid=(B // BM,),
        in_specs=[
            pl.BlockSpec((BM, 2 * EMB), lambda i: (i, 0)),
            pl.BlockSpec((BM, 2 * EMB), lambda i: (i, 0)),
            pl.BlockSpec((BM, 1), lambda i: (i, 0)),
            pl.BlockSpec((BM, 1), lambda i: (i, 0)),
            pl.BlockSpec((2 * EMB, HID), lambda i: (0, 0)),
            pl.BlockSpec((2 * EMB, HID), lambda i: (0, 0)),
            pl.BlockSpec((1, HID), lambda i: (0, 0)),
            pl.BlockSpec((1, HID), lambda i: (0, 0)),
            pl.BlockSpec((1, 1), lambda i: (0, 0)),
        ],
        out_specs=pl.BlockSpec((BM, 1), lambda i: (i, 0)),
        out_shape=jax.ShapeDtypeStruct((B, 1), jnp.float32),
    )(gu, gi, uh, ih, W1uu, W1ii, b1.reshape(1, HID),
      W2.reshape(1, HID), b2.reshape(1, 1))
    return out
